# baseline (speedup 1.0000x reference)
import functools
import math

import jax
import jax.numpy as jnp
from jax import lax
from jax.experimental import pallas as pl
from jax.experimental.pallas import tpu as pltpu
from jax.experimental.pallas import tpu_sc as plsc

_R_MIN = 0.0
_R_MAX = 10.0
_R_BINS = 20
_DR = (_R_MAX - _R_MIN) / _R_BINS

_NC, _NS, _L = 2, 16, 16
_NW = _NC * _NS
_HB = 32


def _bin_index(x, y, lane, lane_slot):
    u = x * x + y * y
    u = jnp.maximum(u, jnp.float32(1e-12))
    yk = plsc.bitcast(
        jnp.int32(0x5F3759DF)
        - lax.shift_right_logical(plsc.bitcast(u, jnp.int32), 1),
        jnp.float32,
    )
    uh = 0.5 * u
    yk = yk * (1.5 - uh * yk * yk)
    yk = yk * (1.5 - uh * yk * yk)
    binf = (u + u) * yk
    binf = jnp.minimum(binf, jnp.float32(_HB - 1))
    k = binf.astype(jnp.int32)
    return lane_slot + ((k + lane) & (_HB - 1))


def _make_kernel(n, block, mbase):
    per_tile = n // _NW
    nblk = per_tile // block
    ngrp = block // _L
    mesh = plsc.VectorSubcoreMesh(
        core_axis_name="c", subcore_axis_name="s",
        num_cores=_NC, num_subcores=_NS,
    )

    @functools.partial(
        pl.kernel,
        mesh=mesh,
        out_type=jax.ShapeDtypeStruct((_NW, _HB), jnp.float32),
        compiler_params=pltpu.CompilerParams(needs_layout_passes=False),
        scratch_types=[
            pltpu.VMEM((2, block), jnp.float32),
            pltpu.VMEM((2, block), jnp.float32),
            pltpu.VMEM((2, block), jnp.float32),
            pltpu.VMEM((_L * _HB,), jnp.float32),
            pltpu.VMEM((_HB,), jnp.float32),
            pltpu.SemaphoreType.DMA,
            pltpu.SemaphoreType.DMA,
        ],
    )
    def hist_kernel(x_hbm, y_hbm, mass_hbm, out_hbm,
                    xv, yv, massv, histv, rowv, sem0, sem1):
        wid = lax.axis_index("s") * _NC + lax.axis_index("c")
        base = wid * per_tile
        lane = lax.iota(jnp.int32, 16)
        lane_slot = lane * _HB
        zero16 = jnp.zeros((_L,), jnp.float32)
        sems = (sem0, sem1)

        for j in range(_HB):
            histv[pl.ds(j * _L, _L)] = zero16

        def issue(b, s):
            start = base + b * block
            pltpu.async_copy(x_hbm.at[pl.ds(start, block)], xv.at[s], sems[s])
            pltpu.async_copy(y_hbm.at[pl.ds(start, block)], yv.at[s], sems[s])
            pltpu.async_copy(mass_hbm.at[pl.ds(mbase + start, block)],
                             massv.at[s], sems[s])

        def drain(b, s):
            start = base + b * block
            pltpu.make_async_copy(
                x_hbm.at[pl.ds(start, block)], xv.at[s], sems[s]).wait()
            pltpu.make_async_copy(
                y_hbm.at[pl.ds(start, block)], yv.at[s], sems[s]).wait()
            pltpu.make_async_copy(
                mass_hbm.at[pl.ds(mbase + start, block)], massv.at[s],
                sems[s]).wait()

        issue(0, 0)
        if nblk > 1:
            issue(1, 1)

        @pl.loop(0, nblk, step=2)
        def _blk(b0):
            for s in range(2):
                b = b0 + s
                drain(b, s)

                @plsc.parallel_loop(0, ngrp, unroll=8)
                def _grp(g):
                    x = xv[s, pl.ds(g * _L, _L)]
                    y = yv[s, pl.ds(g * _L, _L)]
                    m = massv[s, pl.ds(g * _L, _L)]
                    plsc.addupdate_scatter(
                        histv, [_bin_index(x, y, lane, lane_slot)], m)

                @pl.when(b + 2 < nblk)
                def _():
                    issue(b + 2, s)

        acc0 = zero16
        acc1 = zero16
        for l in range(_L):
            acc0 = acc0 + plsc.load_gather(
                histv, [l * _HB + ((lane + l) & (_HB - 1))])
            acc1 = acc1 + plsc.load_gather(
                histv, [l * _HB + ((lane + _L + l) & (_HB - 1))])
        rowv[pl.ds(0, _L)] = acc0
        rowv[pl.ds(_L, _L)] = acc1
        pltpu.sync_copy(rowv, out_hbm.at[wid])

    return hist_kernel


def kernel(positions, masses, area):
    n = positions.shape[0]
    chunks = 8 if n % 8 == 0 else 1
    nc = n // chunks
    block = 8192
    while nc % (_NW * 2 * block) != 0:
        block //= 2
    parts = []
    pos = positions
    for c in range(chunks):
        if c > 0:
            pos = lax.optimization_barrier(pos)
        xs = pos[c * nc:(c + 1) * nc, 0]
        ys = pos[c * nc:(c + 1) * nc, 1]
        part = _make_kernel(nc, block, c * nc)(xs, ys, masses)
        parts.append(part[:, :_R_BINS].sum(axis=0))
    return sum(parts) / area


# speedup vs baseline: 1.0183x; 1.0183x over previous
import functools
import math

import jax
import jax.numpy as jnp
from jax import lax
from jax.experimental import pallas as pl
from jax.experimental.pallas import tpu as pltpu
from jax.experimental.pallas import tpu_sc as plsc

_R_MIN = 0.0
_R_MAX = 10.0
_R_BINS = 20
_DR = (_R_MAX - _R_MIN) / _R_BINS

_NC, _NS, _L = 2, 16, 16
_NW = _NC * _NS
_HB = 32


def _bin_index(x, y, lane, lane_slot):
    u = x * x + y * y
    u = jnp.maximum(u, jnp.float32(1e-12))
    yk = plsc.bitcast(
        jnp.int32(0x5F3759DF)
        - lax.shift_right_logical(plsc.bitcast(u, jnp.int32), 1),
        jnp.float32,
    )
    uh = 0.5 * u
    yk = yk * (1.5 - uh * yk * yk)
    yk = yk * (1.5 - uh * yk * yk)
    binf = (u + u) * yk
    binf = jnp.minimum(binf, jnp.float32(_HB - 1))
    k = binf.astype(jnp.int32)
    return lane_slot + ((k + lane) & (_HB - 1))


def _make_kernel(n, block, mbase):
    per_tile = n // _NW
    nblk = per_tile // block
    ngrp = block // _L
    mesh = plsc.VectorSubcoreMesh(
        core_axis_name="c", subcore_axis_name="s",
        num_cores=_NC, num_subcores=_NS,
    )

    @functools.partial(
        pl.kernel,
        mesh=mesh,
        out_type=jax.ShapeDtypeStruct((_NW, _HB), jnp.float32),
        compiler_params=pltpu.CompilerParams(needs_layout_passes=False),
        scratch_types=[
            pltpu.VMEM((2, block), jnp.float32),
            pltpu.VMEM((2, block), jnp.float32),
            pltpu.VMEM((2, block), jnp.float32),
            pltpu.VMEM((_L * _HB,), jnp.float32),
            pltpu.VMEM((_HB,), jnp.float32),
            pltpu.SemaphoreType.DMA,
            pltpu.SemaphoreType.DMA,
        ],
    )
    def hist_kernel(x_hbm, y_hbm, mass_hbm, out_hbm,
                    xv, yv, massv, histv, rowv, sem0, sem1):
        wid = lax.axis_index("s") * _NC + lax.axis_index("c")
        base = wid * per_tile
        lane = lax.iota(jnp.int32, 16)
        lane_slot = lane * _HB
        zero16 = jnp.zeros((_L,), jnp.float32)
        sems = (sem0, sem1)

        for j in range(_HB):
            histv[pl.ds(j * _L, _L)] = zero16

        def issue(b, s):
            start = base + b * block
            pltpu.async_copy(x_hbm.at[pl.ds(start, block)], xv.at[s], sems[s])
            pltpu.async_copy(y_hbm.at[pl.ds(start, block)], yv.at[s], sems[s])
            pltpu.async_copy(mass_hbm.at[pl.ds(mbase + start, block)],
                             massv.at[s], sems[s])

        def drain(b, s):
            start = base + b * block
            pltpu.make_async_copy(
                x_hbm.at[pl.ds(start, block)], xv.at[s], sems[s]).wait()
            pltpu.make_async_copy(
                y_hbm.at[pl.ds(start, block)], yv.at[s], sems[s]).wait()
            pltpu.make_async_copy(
                mass_hbm.at[pl.ds(mbase + start, block)], massv.at[s],
                sems[s]).wait()

        issue(0, 0)
        if nblk > 1:
            issue(1, 1)

        @pl.loop(0, nblk, step=2)
        def _blk(b0):
            for s in range(2):
                b = b0 + s
                drain(b, s)

                @plsc.parallel_loop(0, ngrp, unroll=8)
                def _grp(g):
                    x = xv[s, pl.ds(g * _L, _L)]
                    y = yv[s, pl.ds(g * _L, _L)]
                    m = massv[s, pl.ds(g * _L, _L)]
                    plsc.addupdate_scatter(
                        histv, [_bin_index(x, y, lane, lane_slot)], m)

                @pl.when(b + 2 < nblk)
                def _():
                    issue(b + 2, s)

        acc0 = zero16
        acc1 = zero16
        for l in range(_L):
            acc0 = acc0 + plsc.load_gather(
                histv, [l * _HB + ((lane + l) & (_HB - 1))])
            acc1 = acc1 + plsc.load_gather(
                histv, [l * _HB + ((lane + _L + l) & (_HB - 1))])
        rowv[pl.ds(0, _L)] = acc0
        rowv[pl.ds(_L, _L)] = acc1
        pltpu.sync_copy(rowv, out_hbm.at[wid])

    return hist_kernel


def kernel(positions, masses, area):
    n = positions.shape[0]
    if n % 8 == 0:
        bounds = [0, n // 8, n // 2, 7 * n // 8, n]
    else:
        bounds = [0, n]
    parts = []
    pos = positions
    for c in range(len(bounds) - 1):
        lo, hi = bounds[c], bounds[c + 1]
        nc = hi - lo
        block = 8192
        while nc % (_NW * 2 * block) != 0:
            block //= 2
        if c > 0:
            pos = lax.optimization_barrier(pos)
        xs = pos[lo:hi, 0]
        ys = pos[lo:hi, 1]
        part = _make_kernel(nc, block, lo)(xs, ys, masses)
        parts.append(part[:, :_R_BINS].sum(axis=0))
    return sum(parts) / area


# speedup vs baseline: 1.0754x; 1.0560x over previous
import functools
import math

import jax
import jax.numpy as jnp
from jax import lax
from jax.experimental import pallas as pl
from jax.experimental.pallas import tpu as pltpu
from jax.experimental.pallas import tpu_sc as plsc

_R_MIN = 0.0
_R_MAX = 10.0
_R_BINS = 20
_DR = (_R_MAX - _R_MIN) / _R_BINS

_NC, _NS, _L = 2, 16, 16
_NW = _NC * _NS
_HB = 32


def _bin_index(x, y, lane, lane_slot):
    u = x * x + y * y
    u = jnp.maximum(u, jnp.float32(1e-12))
    yk = plsc.bitcast(
        jnp.int32(0x5F3759DF)
        - lax.shift_right_logical(plsc.bitcast(u, jnp.int32), 1),
        jnp.float32,
    )
    uh = 0.5 * u
    yk = yk * (1.5 - uh * yk * yk)
    yk = yk * (1.5 - uh * yk * yk)
    binf = (u + u) * yk
    binf = jnp.minimum(binf, jnp.float32(_HB - 1))
    k = binf.astype(jnp.int32)
    return lane_slot + ((k + lane) & (_HB - 1))


def _make_kernel(n, block, mbase):
    per_tile = n // _NW
    nblk = per_tile // block
    ngrp = block // _L
    mesh = plsc.VectorSubcoreMesh(
        core_axis_name="c", subcore_axis_name="s",
        num_cores=_NC, num_subcores=_NS,
    )

    @functools.partial(
        pl.kernel,
        mesh=mesh,
        out_type=jax.ShapeDtypeStruct((_NW, _HB), jnp.float32),
        compiler_params=pltpu.CompilerParams(needs_layout_passes=False),
        scratch_types=[
            pltpu.VMEM((2, block), jnp.float32),
            pltpu.VMEM((2, block), jnp.float32),
            pltpu.VMEM((2, block), jnp.float32),
            pltpu.VMEM((_L * _HB,), jnp.float32),
            pltpu.VMEM((_HB,), jnp.float32),
            pltpu.SemaphoreType.DMA,
            pltpu.SemaphoreType.DMA,
        ],
    )
    def hist_kernel(x_hbm, y_hbm, mass_hbm, out_hbm,
                    xv, yv, massv, histv, rowv, sem0, sem1):
        wid = lax.axis_index("s") * _NC + lax.axis_index("c")
        base = wid * per_tile
        lane = lax.iota(jnp.int32, 16)
        lane_slot = lane * _HB
        zero16 = jnp.zeros((_L,), jnp.float32)
        sems = (sem0, sem1)

        for j in range(_HB):
            histv[pl.ds(j * _L, _L)] = zero16

        def issue(b, s):
            start = base + b * block
            pltpu.async_copy(x_hbm.at[pl.ds(start, block)], xv.at[s], sems[s])
            pltpu.async_copy(y_hbm.at[pl.ds(start, block)], yv.at[s], sems[s])
            pltpu.async_copy(mass_hbm.at[pl.ds(mbase + start, block)],
                             massv.at[s], sems[s])

        def drain(b, s):
            start = base + b * block
            pltpu.make_async_copy(
                x_hbm.at[pl.ds(start, block)], xv.at[s], sems[s]).wait()
            pltpu.make_async_copy(
                y_hbm.at[pl.ds(start, block)], yv.at[s], sems[s]).wait()
            pltpu.make_async_copy(
                mass_hbm.at[pl.ds(mbase + start, block)], massv.at[s],
                sems[s]).wait()

        issue(0, 0)
        if nblk > 1:
            issue(1, 1)

        @pl.loop(0, nblk, step=2)
        def _blk(b0):
            for s in range(2):
                b = b0 + s
                drain(b, s)

                @plsc.parallel_loop(0, ngrp, unroll=8)
                def _grp(g):
                    x = xv[s, pl.ds(g * _L, _L)]
                    y = yv[s, pl.ds(g * _L, _L)]
                    m = massv[s, pl.ds(g * _L, _L)]
                    plsc.addupdate_scatter(
                        histv, [_bin_index(x, y, lane, lane_slot)], m)

                @pl.when(b + 2 < nblk)
                def _():
                    issue(b + 2, s)

        acc0 = zero16
        acc1 = zero16
        for l in range(_L):
            acc0 = acc0 + plsc.load_gather(
                histv, [l * _HB + ((lane + l) & (_HB - 1))])
            acc1 = acc1 + plsc.load_gather(
                histv, [l * _HB + ((lane + _L + l) & (_HB - 1))])
        rowv[pl.ds(0, _L)] = acc0
        rowv[pl.ds(_L, _L)] = acc1
        pltpu.sync_copy(rowv, out_hbm.at[wid])

    return hist_kernel


def kernel(positions, masses, area):
    n = positions.shape[0]
    if n % 4 == 0:
        bounds = [0, n // 4, n // 2, 3 * n // 4, n]
    else:
        bounds = [0, n]
    parts = []
    pos = positions
    for c in range(len(bounds) - 1):
        lo, hi = bounds[c], bounds[c + 1]
        nc = hi - lo
        block = 8192
        while nc % (_NW * 2 * block) != 0:
            block //= 2
        if c > 0:
            pos = lax.optimization_barrier(pos)
        xs = pos[lo:hi, 0]
        ys = pos[lo:hi, 1]
        part = _make_kernel(nc, block, lo)(xs, ys, masses)
        parts.append(part[:, :_R_BINS].sum(axis=0))
    return sum(parts) / area
